# Initial kernel scaffold; baseline (speedup 1.0000x reference)
import jax
import jax.numpy as jnp
from jax.experimental import pallas as pl


def kernel(x, edge_index, edge_attr, batch, params):
    raise NotImplementedError("write your pallas kernel here")


# speedup vs baseline: 2.6388x; 2.6388x over previous
import functools

import jax
import jax.numpy as jnp
from jax import lax
from jax.experimental import pallas as pl
from jax.experimental.pallas import tpu as pltpu
from jax.experimental.pallas import tpu_sc as plsc

N = 10000
E = 320000
DN = 128
DE = 16
H = 128
B = 16

NP = 10240
NW = 32
K = 128
CH = (E + NW * K - 1) // (NW * K)
EP = NW * K * CH
EPW = EP // NW
RPS = NP // 16


@functools.lru_cache(maxsize=None)
def _make_msgpass():
    mesh = plsc.VectorSubcoreMesh(core_axis_name="c", subcore_axis_name="s")

    @functools.partial(
        pl.kernel,
        mesh=mesh,
        out_type=jax.ShapeDtypeStruct((2, NP, H), jnp.float32),
        scratch_types=[
            pltpu.VMEM((K,), jnp.int32),
            pltpu.VMEM((CH, K), jnp.int32),
            pltpu.VMEM((K, H), jnp.float32),
            pltpu.VMEM((K, H), jnp.float32),
            pltpu.VMEM_SHARED((NP, H), jnp.float32),
            pltpu.SemaphoreType.DMA,
        ],
    )
    def msgpass(node_hbm, edge_hbm, src_hbm, dst_hbm, out_hbm,
                src_v, dst_v, gat_v, edg_v, acc_sh, sem):
        c = lax.axis_index("c")
        s = lax.axis_index("s")
        wid = s * 2 + c

        pltpu.sync_copy(dst_hbm.at[wid], dst_v)

        def zrow(r, carry):
            for j in range(H // 16):
                gat_v[r, pl.ds(j * 16, 16)] = jnp.zeros((16,), jnp.float32)
            return carry
        lax.fori_loop(0, K, zrow, 0)
        for i in range(RPS // K):
            pltpu.sync_copy(gat_v, acc_sh.at[pl.ds(s * RPS + i * K, K)])
        plsc.subcore_barrier()

        def chunk(i, carry):
            pltpu.sync_copy(src_hbm.at[wid].at[i], src_v)
            pltpu.async_copy(node_hbm.at[src_v], gat_v, sem).wait()
            pltpu.sync_copy(edge_hbm.at[pl.ds(wid * EPW + i * K, K)], edg_v)

            def row(r, rc):
                for j in range(H // 16):
                    sl = pl.ds(j * 16, 16)
                    gat_v[r, sl] = jnp.maximum(gat_v[r, sl] + edg_v[r, sl],
                                               0.0)
                return rc
            lax.fori_loop(0, K, row, 0)

            pltpu.sync_copy(gat_v, acc_sh.at[dst_v.at[i]], add=True)
            return carry
        lax.fori_loop(0, CH, chunk, 0)

        plsc.subcore_barrier()
        for i in range(RPS // K):
            sl = pl.ds(s * RPS + i * K, K)
            pltpu.sync_copy(acc_sh.at[sl], out_hbm.at[c].at[sl])

    return msgpass


def _dot(a, b):
    return jax.lax.dot_general(a, b, (((1,), (0,)), ((), ())),
                               preferred_element_type=jnp.float32)


def _node_encode_body(x_ref, w_ref, b_ref, o_ref):
    o_ref[...] = _dot(x_ref[...], w_ref[...]) + b_ref[...]


def _edge_body(ea_ref, w0_ref, b0_ref, w1_ref, b1_ref, w2_ref, b2_ref,
               e0_ref, e1_ref):
    e0 = _dot(ea_ref[...], w0_ref[...]) + b0_ref[...]
    e0_ref[...] = e0
    t = jnp.maximum(_dot(e0, w1_ref[...]) + b1_ref[...], 0.0)
    e1_ref[...] = _dot(t, w2_ref[...]) + b2_ref[...]


def _update_body(h_ref, a0_ref, a1_ref, w1_ref, b1_ref, w2_ref, b2_ref,
                 wih_ref, bih_ref, whh_ref, bhh_ref, eps_ref, o_ref):
    h = h_ref[...]
    u = (1.0 + eps_ref[0, 0]) * h + a0_ref[...] + a1_ref[...]
    t = jnp.maximum(_dot(u, w1_ref[...]) + b1_ref[...], 0.0)
    m = _dot(t, w2_ref[...]) + b2_ref[...]
    gi = _dot(m, wih_ref[...]) + bih_ref[...]
    gh = _dot(h, whh_ref[...]) + bhh_ref[...]
    r = jax.nn.sigmoid(gi[:, 0:H] + gh[:, 0:H])
    z = jax.nn.sigmoid(gi[:, H:2 * H] + gh[:, H:2 * H])
    n = jnp.tanh(gi[:, 2 * H:3 * H] + r * gh[:, 2 * H:3 * H])
    o_ref[...] = (1.0 - z) * n + z * h


def _set2set_body(h_ref, bat_ref, wih_ref, bih_ref, whh_ref, bhh_ref,
                  f1w_ref, f1b_ref, f2w_ref, f2b_ref, o_ref):
    h = h_ref[...]
    seg = jax.lax.broadcasted_iota(jnp.int32, (B, NP), 0)
    mask = bat_ref[...] == seg
    hs = jnp.zeros((B, H), jnp.float32)
    cs = jnp.zeros((B, H), jnp.float32)
    qs = jnp.zeros((B, 2 * H), jnp.float32)
    for _ in range(3):
        gates = (_dot(qs, wih_ref[...]) + bih_ref[...]
                 + _dot(hs, whh_ref[...]) + bhh_ref[...])
        gi = gates[:, 0:H]
        gf = gates[:, H:2 * H]
        gg = gates[:, 2 * H:3 * H]
        go = gates[:, 3 * H:4 * H]
        cs = jax.nn.sigmoid(gf) * cs + jax.nn.sigmoid(gi) * jnp.tanh(gg)
        hs = jax.nn.sigmoid(go) * jnp.tanh(cs)
        st = jax.lax.dot_general(hs, h, (((1,), (1,)), ((), ())),
                                 preferred_element_type=jnp.float32)
        sm = jnp.where(mask, st, -jnp.inf)
        e_max = jnp.maximum(jnp.max(sm, axis=1, keepdims=True), -1e30)
        a = jnp.where(mask, jnp.exp(st - e_max), 0.0)
        denom = jnp.sum(a, axis=1, keepdims=True)
        attn = a / (denom + 1e-16)
        r = _dot(attn, h)
        qs = jnp.concatenate([hs, r], axis=1)
    z1 = jnp.maximum(_dot(qs, f1w_ref[...]) + f1b_ref[...], 0.0)
    o_ref[...] = _dot(z1, f2w_ref[...]) + f2b_ref[...]


def _full(shape):
    return pl.BlockSpec(shape, lambda i: (0, 0))


def _rows(blk, d):
    return pl.BlockSpec((blk, d), lambda i: (i, 0))


def kernel(x, edge_index, edge_attr, batch, params):
    f32 = jnp.float32
    src = edge_index[0].astype(jnp.int32)
    dst = edge_index[1].astype(jnp.int32)
    pe = EP - E
    src2 = jnp.concatenate([src, jnp.zeros((pe,), jnp.int32)]).reshape(
        NW, CH, K)
    dst2 = jnp.concatenate([dst, jnp.full((pe,), N, jnp.int32)]).reshape(
        NW, CH, K)
    x_p = jnp.pad(x, ((0, NP - N), (0, 0)))
    ea_p = jnp.pad(edge_attr, ((0, pe), (0, 0)))
    bat_p = jnp.pad(batch.astype(jnp.int32), (0, NP - N),
                    constant_values=B).reshape(1, NP)

    p = params
    row1 = lambda a: a.reshape(1, -1)

    node0 = pl.pallas_call(
        _node_encode_body,
        grid=(NP // 2048,),
        in_specs=[_rows(2048, DN), _full((DN, H)), _full((1, H))],
        out_specs=_rows(2048, H),
        out_shape=jax.ShapeDtypeStruct((NP, H), f32),
    )(x_p, p["node_lin0"]["W"], row1(p["node_lin0"]["b"]))

    eblk = 2048
    e0, e1 = pl.pallas_call(
        _edge_body,
        grid=(EP // eblk,),
        in_specs=[_rows(eblk, DE), _full((DE, H)), _full((1, H)),
                  _full((H, H)), _full((1, H)), _full((H, H)), _full((1, H))],
        out_specs=[_rows(eblk, H), _rows(eblk, H)],
        out_shape=[jax.ShapeDtypeStruct((EP, H), f32),
                   jax.ShapeDtypeStruct((EP, H), f32)],
    )(ea_p, p["edge_lin0"]["W"], row1(p["edge_lin0"]["b"]),
      p["edge_mlp"][0]["mlp1"]["W"], row1(p["edge_mlp"][0]["mlp1"]["b"]),
      p["edge_mlp"][0]["mlp2"]["W"], row1(p["edge_mlp"][0]["mlp2"]["b"]))

    gru = p["gru"]
    wih_t = gru["W_ih"].T
    whh_t = gru["W_hh"].T
    bih = row1(gru["b_ih"])
    bhh = row1(gru["b_hh"])

    h = node0
    for layer, edge_out in ((0, e0), (1, e1)):
        gp = p["gin"][layer]
        aggs = _make_msgpass()(h, edge_out, src2, dst2)
        ublk = 2048
        h = pl.pallas_call(
            _update_body,
            grid=(NP // ublk,),
            in_specs=[_rows(ublk, H), _rows(ublk, H), _rows(ublk, H),
                      _full((H, H)), _full((1, H)), _full((H, H)),
                      _full((1, H)), _full((H, 3 * H)), _full((1, 3 * H)),
                      _full((H, 3 * H)), _full((1, 3 * H)),
                      pl.BlockSpec(memory_space=pltpu.SMEM)],
            out_specs=_rows(ublk, H),
            out_shape=jax.ShapeDtypeStruct((NP, H), f32),
        )(h, aggs[0], aggs[1],
          gp["mlp1"]["W"], row1(gp["mlp1"]["b"]),
          gp["mlp2"]["W"], row1(gp["mlp2"]["b"]),
          wih_t, bih, whh_t, bhh, gp["eps"].reshape(1, 1))

    lstm = p["lstm"]
    f2w = jnp.zeros((H, H), f32).at[:, 0].set(p["fc2"]["W"][:, 0])
    f2b = jnp.zeros((1, H), f32).at[0, 0].set(p["fc2"]["b"][0])
    out = pl.pallas_call(
        _set2set_body,
        grid=(1,),
        in_specs=[_rows(NP, H), _full((1, NP)),
                  _full((2 * H, 4 * H)), _full((1, 4 * H)),
                  _full((H, 4 * H)), _full((1, 4 * H)),
                  _full((2 * H, H)), _full((1, H)),
                  _full((H, H)), _full((1, H))],
        out_specs=_rows(B, H),
        out_shape=jax.ShapeDtypeStruct((B, H), f32),
    )(h, bat_p, lstm["W_ih"].T, row1(lstm["b_ih"]),
      lstm["W_hh"].T, row1(lstm["b_hh"]),
      p["fc1"]["W"], row1(p["fc1"]["b"]), f2w, f2b)
    return out[:, 0]


# speedup vs baseline: 4.5720x; 1.7326x over previous
import functools

import jax
import jax.numpy as jnp
from jax import lax
from jax.experimental import pallas as pl
from jax.experimental.pallas import tpu as pltpu
from jax.experimental.pallas import tpu_sc as plsc

N = 10000
E = 320000
DN = 128
DE = 16
H = 128
B = 16

NP = 10240
NW = 32
K = 56
CH = (E + NW * K - 1) // (NW * K)
EP = NW * K * CH
EPW = EP // NW
RPS = NP // 16
NB = 3


@functools.lru_cache(maxsize=None)
def _make_msgpass():
    mesh = plsc.VectorSubcoreMesh(core_axis_name="c", subcore_axis_name="s")

    @functools.partial(
        pl.kernel,
        mesh=mesh,
        out_type=jax.ShapeDtypeStruct((2, NP, H), jnp.float32),
        scratch_types=[
            pltpu.VMEM((2 * NB, K), jnp.int32),
            pltpu.VMEM((2 * NB, K), jnp.int32),
            pltpu.VMEM((NB, K, H), jnp.float32),
            pltpu.VMEM((NB, K, H), jnp.float32),
            pltpu.VMEM_SHARED((NP, H), jnp.float32),
        ] + [pltpu.SemaphoreType.DMA] * (4 * NB),
    )
    def msgpass(node_hbm, edge_hbm, src_hbm, dst_hbm, out_hbm,
                src_v, dst_v, gat_v, edg_v, acc_sh, *sems):
        c = lax.axis_index("c")
        s = lax.axis_index("s")
        wid = s * 2 + c
        gsem = sems[0:NB]
        esem = sems[NB:2 * NB]
        isem = sems[2 * NB:3 * NB]
        ssem = sems[3 * NB:4 * NB]

        def zrow(r, carry):
            for j in range(H // 16):
                gat_v[0, r, pl.ds(j * 16, 16)] = jnp.zeros((16,),
                                                           jnp.float32)
            return carry
        lax.fori_loop(0, K, zrow, 0)
        base0 = s * RPS
        for i in range(RPS // K):
            pltpu.sync_copy(gat_v.at[0], acc_sh.at[pl.ds(base0 + i * K, K)])
        rem = RPS - (RPS // K) * K
        if rem:
            pltpu.sync_copy(gat_v.at[0].at[pl.ds(0, rem)],
                            acc_sh.at[pl.ds(base0 + (RPS // K) * K, rem)])
        plsc.subcore_barrier()

        def issue_idx(ci, bi):
            pltpu.async_copy(src_hbm.at[wid, ci], src_v.at[bi],
                             isem[bi % NB])
            pltpu.async_copy(dst_hbm.at[wid, ci], dst_v.at[bi],
                             isem[bi % NB])

        def wait_idx(ci, bi):
            pltpu.make_async_copy(src_hbm.at[wid, ci], src_v.at[bi],
                                  isem[bi % NB]).wait()
            pltpu.make_async_copy(dst_hbm.at[wid, ci], dst_v.at[bi],
                                  isem[bi % NB]).wait()

        def issue_edge(ci, b):
            pltpu.async_copy(edge_hbm.at[pl.ds(wid * EPW + ci * K, K)],
                             edg_v.at[b], esem[b])

        def wait_edge(ci, b):
            pltpu.make_async_copy(edge_hbm.at[pl.ds(wid * EPW + ci * K, K)],
                                  edg_v.at[b], esem[b]).wait()

        def issue_gather(b, bi):
            pltpu.async_copy(node_hbm.at[src_v.at[bi]], gat_v.at[b],
                             gsem[b])

        def wait_gather(b, bi):
            pltpu.make_async_copy(node_hbm.at[src_v.at[bi]], gat_v.at[b],
                                  gsem[b]).wait()

        def issue_scatter(b, bi):
            pltpu.async_copy(gat_v.at[b], acc_sh.at[dst_v.at[bi]], ssem[b],
                             add=True)

        def wait_scatter(b, bi):
            pltpu.make_async_copy(gat_v.at[b], acc_sh.at[dst_v.at[bi]],
                                  ssem[b]).wait()

        for pch in range(NB):
            if pch < CH:
                issue_idx(pch, pch)
                issue_edge(pch, pch)
        wait_idx(0, 0)
        issue_gather(0, 0)

        def ring(g, carry):
            for b6 in range(2 * NB):
                ci = g * 2 * NB + b6
                b = b6 % NB
                bp1 = (b + 1) % NB
                bi1 = (b6 + 1) % (2 * NB)
                bi3 = (b6 + NB) % (2 * NB)

                @pl.when(ci < CH)
                def _section():
                    wait_gather(b, b6)
                    wait_edge(ci, b)

                    @pl.when(ci + 1 < CH)
                    def _():
                        wait_idx(ci + 1, bi1)

                    @pl.when(ci >= 2)
                    def _():
                        wait_scatter(bp1, (b6 + 2 * NB - 2) % (2 * NB))

                    @pl.when(ci + 1 < CH)
                    def _():
                        issue_gather(bp1, bi1)

                    def row(r, rc):
                        for j in range(H // 16):
                            sl = pl.ds(j * 16, 16)
                            gat_v[b, r, sl] = jnp.maximum(
                                gat_v[b, r, sl] + edg_v[b, r, sl], 0.0)
                        return rc
                    lax.fori_loop(0, K, row, 0)

                    issue_scatter(b, b6)

                    @pl.when(ci + NB < CH)
                    def _():
                        issue_idx(ci + NB, bi3)
                        issue_edge(ci + NB, b)
            return carry
        lax.fori_loop(0, (CH + 2 * NB - 1) // (2 * NB), ring, 0)

        wait_scatter((CH - 2) % NB, (CH - 2) % (2 * NB))
        wait_scatter((CH - 1) % NB, (CH - 1) % (2 * NB))
        plsc.subcore_barrier()
        pltpu.sync_copy(acc_sh.at[pl.ds(base0, RPS)],
                        out_hbm.at[c].at[pl.ds(base0, RPS)])

    return msgpass


def _dot(a, b):
    return jax.lax.dot_general(a, b, (((1,), (0,)), ((), ())),
                               preferred_element_type=jnp.float32)


def _node_encode_body(x_ref, w_ref, b_ref, o_ref):
    o_ref[...] = _dot(x_ref[...], w_ref[...]) + b_ref[...]


def _edge_body(ea_ref, w0_ref, b0_ref, w1_ref, b1_ref, w2_ref, b2_ref,
               e0_ref, e1_ref):
    e0 = _dot(ea_ref[...], w0_ref[...]) + b0_ref[...]
    e0_ref[...] = e0
    t = jnp.maximum(_dot(e0, w1_ref[...]) + b1_ref[...], 0.0)
    e1_ref[...] = _dot(t, w2_ref[...]) + b2_ref[...]


def _update_body(h_ref, a0_ref, a1_ref, w1_ref, b1_ref, w2_ref, b2_ref,
                 wih_ref, bih_ref, whh_ref, bhh_ref, eps_ref, o_ref):
    h = h_ref[...]
    u = (1.0 + eps_ref[0, 0]) * h + a0_ref[...] + a1_ref[...]
    t = jnp.maximum(_dot(u, w1_ref[...]) + b1_ref[...], 0.0)
    m = _dot(t, w2_ref[...]) + b2_ref[...]
    gi = _dot(m, wih_ref[...]) + bih_ref[...]
    gh = _dot(h, whh_ref[...]) + bhh_ref[...]
    r = jax.nn.sigmoid(gi[:, 0:H] + gh[:, 0:H])
    z = jax.nn.sigmoid(gi[:, H:2 * H] + gh[:, H:2 * H])
    n = jnp.tanh(gi[:, 2 * H:3 * H] + r * gh[:, 2 * H:3 * H])
    o_ref[...] = (1.0 - z) * n + z * h


def _set2set_body(h_ref, bat_ref, wih_ref, bih_ref, whh_ref, bhh_ref,
                  f1w_ref, f1b_ref, f2w_ref, f2b_ref, o_ref):
    h = h_ref[...]
    seg = jax.lax.broadcasted_iota(jnp.int32, (B, NP), 0)
    mask = bat_ref[...] == seg
    hs = jnp.zeros((B, H), jnp.float32)
    cs = jnp.zeros((B, H), jnp.float32)
    qs = jnp.zeros((B, 2 * H), jnp.float32)
    for _ in range(3):
        gates = (_dot(qs, wih_ref[...]) + bih_ref[...]
                 + _dot(hs, whh_ref[...]) + bhh_ref[...])
        gi = gates[:, 0:H]
        gf = gates[:, H:2 * H]
        gg = gates[:, 2 * H:3 * H]
        go = gates[:, 3 * H:4 * H]
        cs = jax.nn.sigmoid(gf) * cs + jax.nn.sigmoid(gi) * jnp.tanh(gg)
        hs = jax.nn.sigmoid(go) * jnp.tanh(cs)
        st = jax.lax.dot_general(hs, h, (((1,), (1,)), ((), ())),
                                 preferred_element_type=jnp.float32)
        sm = jnp.where(mask, st, -jnp.inf)
        e_max = jnp.maximum(jnp.max(sm, axis=1, keepdims=True), -1e30)
        a = jnp.where(mask, jnp.exp(st - e_max), 0.0)
        denom = jnp.sum(a, axis=1, keepdims=True)
        attn = a / (denom + 1e-16)
        r = _dot(attn, h)
        qs = jnp.concatenate([hs, r], axis=1)
    z1 = jnp.maximum(_dot(qs, f1w_ref[...]) + f1b_ref[...], 0.0)
    o_ref[...] = _dot(z1, f2w_ref[...]) + f2b_ref[...]


def _full(shape):
    return pl.BlockSpec(shape, lambda i: (0, 0))


def _rows(blk, d):
    return pl.BlockSpec((blk, d), lambda i: (i, 0))


def kernel(x, edge_index, edge_attr, batch, params):
    f32 = jnp.float32
    src = edge_index[0].astype(jnp.int32)
    dst = edge_index[1].astype(jnp.int32)
    pe = EP - E
    src2 = jnp.concatenate([src, jnp.zeros((pe,), jnp.int32)]).reshape(
        NW, CH, K)
    dst2 = jnp.concatenate([dst, jnp.full((pe,), N, jnp.int32)]).reshape(
        NW, CH, K)
    x_p = jnp.pad(x, ((0, NP - N), (0, 0)))
    ea_p = jnp.pad(edge_attr, ((0, pe), (0, 0)))
    bat_p = jnp.pad(batch.astype(jnp.int32), (0, NP - N),
                    constant_values=B).reshape(1, NP)

    p = params
    row1 = lambda a: a.reshape(1, -1)

    node0 = pl.pallas_call(
        _node_encode_body,
        grid=(NP // 2048,),
        in_specs=[_rows(2048, DN), _full((DN, H)), _full((1, H))],
        out_specs=_rows(2048, H),
        out_shape=jax.ShapeDtypeStruct((NP, H), f32),
    )(x_p, p["node_lin0"]["W"], row1(p["node_lin0"]["b"]))

    eblk = 2048
    e0, e1 = pl.pallas_call(
        _edge_body,
        grid=(EP // eblk,),
        in_specs=[_rows(eblk, DE), _full((DE, H)), _full((1, H)),
                  _full((H, H)), _full((1, H)), _full((H, H)), _full((1, H))],
        out_specs=[_rows(eblk, H), _rows(eblk, H)],
        out_shape=[jax.ShapeDtypeStruct((EP, H), f32),
                   jax.ShapeDtypeStruct((EP, H), f32)],
    )(ea_p, p["edge_lin0"]["W"], row1(p["edge_lin0"]["b"]),
      p["edge_mlp"][0]["mlp1"]["W"], row1(p["edge_mlp"][0]["mlp1"]["b"]),
      p["edge_mlp"][0]["mlp2"]["W"], row1(p["edge_mlp"][0]["mlp2"]["b"]))

    gru = p["gru"]
    wih_t = gru["W_ih"].T
    whh_t = gru["W_hh"].T
    bih = row1(gru["b_ih"])
    bhh = row1(gru["b_hh"])

    h = node0
    for layer, edge_out in ((0, e0), (1, e1)):
        gp = p["gin"][layer]
        aggs = _make_msgpass()(h, edge_out, src2, dst2)
        ublk = 2048
        h = pl.pallas_call(
            _update_body,
            grid=(NP // ublk,),
            in_specs=[_rows(ublk, H), _rows(ublk, H), _rows(ublk, H),
                      _full((H, H)), _full((1, H)), _full((H, H)),
                      _full((1, H)), _full((H, 3 * H)), _full((1, 3 * H)),
                      _full((H, 3 * H)), _full((1, 3 * H)),
                      pl.BlockSpec(memory_space=pltpu.SMEM)],
            out_specs=_rows(ublk, H),
            out_shape=jax.ShapeDtypeStruct((NP, H), f32),
        )(h, aggs[0], aggs[1],
          gp["mlp1"]["W"], row1(gp["mlp1"]["b"]),
          gp["mlp2"]["W"], row1(gp["mlp2"]["b"]),
          wih_t, bih, whh_t, bhh, gp["eps"].reshape(1, 1))

    lstm = p["lstm"]
    f2w = jnp.zeros((H, H), f32).at[:, 0].set(p["fc2"]["W"][:, 0])
    f2b = jnp.zeros((1, H), f32).at[0, 0].set(p["fc2"]["b"][0])
    out = pl.pallas_call(
        _set2set_body,
        grid=(1,),
        in_specs=[_rows(NP, H), _full((1, NP)),
                  _full((2 * H, 4 * H)), _full((1, 4 * H)),
                  _full((H, 4 * H)), _full((1, 4 * H)),
                  _full((2 * H, H)), _full((1, H)),
                  _full((H, H)), _full((1, H))],
        out_specs=_rows(B, H),
        out_shape=jax.ShapeDtypeStruct((B, H), f32),
    )(h, bat_p, lstm["W_ih"].T, row1(lstm["b_ih"]),
      lstm["W_hh"].T, row1(lstm["b_hh"]),
      p["fc1"]["W"], row1(p["fc1"]["b"]), f2w, f2b)
    return out[:, 0]


# speedup vs baseline: 4.6001x; 1.0061x over previous
import functools

import jax
import jax.numpy as jnp
from jax import lax
from jax.experimental import pallas as pl
from jax.experimental.pallas import tpu as pltpu
from jax.experimental.pallas import tpu_sc as plsc

N = 10000
E = 320000
DN = 128
DE = 16
H = 128
B = 16

NP = 10240
NW = 32
K = 40
CH = E // (NW * K)
EPW = E // NW
RPS = NP // 16
NB = 3


@functools.lru_cache(maxsize=None)
def _make_msgpass():
    mesh = plsc.VectorSubcoreMesh(core_axis_name="c", subcore_axis_name="s")

    @functools.partial(
        pl.kernel,
        mesh=mesh,
        out_type=jax.ShapeDtypeStruct((2, NP, H), jnp.float32),
        scratch_types=[
            pltpu.VMEM((2 * NB, K), jnp.int32),
            pltpu.VMEM((2 * NB, K), jnp.int32),
            pltpu.VMEM((NB, K, H), jnp.float32),
            pltpu.VMEM((NB, K, H), jnp.float32),
            pltpu.VMEM_SHARED((NP, H), jnp.float32),
        ] + [pltpu.SemaphoreType.DMA] * (4 * NB),
    )
    def msgpass(node_hbm, edge_hbm, src_hbm, dst_hbm, out_hbm,
                src_v, dst_v, gat_v, edg_v, acc_sh, *sems):
        c = lax.axis_index("c")
        s = lax.axis_index("s")
        wid = s * 2 + c
        gsem = sems[0:NB]
        esem = sems[NB:2 * NB]
        isem = sems[2 * NB:3 * NB]
        ssem = sems[3 * NB:4 * NB]

        def zrow(r, carry):
            for j in range(H // 16):
                gat_v[0, r, pl.ds(j * 16, 16)] = jnp.zeros((16,),
                                                           jnp.float32)
            return carry
        lax.fori_loop(0, K, zrow, 0)
        base0 = s * RPS
        for i in range(RPS // K):
            pltpu.sync_copy(gat_v.at[0], acc_sh.at[pl.ds(base0 + i * K, K)])
        rem = RPS - (RPS // K) * K
        if rem:
            pltpu.sync_copy(gat_v.at[0].at[pl.ds(0, rem)],
                            acc_sh.at[pl.ds(base0 + (RPS // K) * K, rem)])
        plsc.subcore_barrier()

        def issue_idx(ci, bi):
            pltpu.async_copy(src_hbm.at[wid, ci], src_v.at[bi],
                             isem[bi % NB])
            pltpu.async_copy(dst_hbm.at[wid, ci], dst_v.at[bi],
                             isem[bi % NB])

        def wait_idx(ci, bi):
            pltpu.make_async_copy(src_hbm.at[wid, ci], src_v.at[bi],
                                  isem[bi % NB]).wait()
            pltpu.make_async_copy(dst_hbm.at[wid, ci], dst_v.at[bi],
                                  isem[bi % NB]).wait()

        def issue_edge(ci, b):
            pltpu.async_copy(edge_hbm.at[pl.ds(wid * EPW + ci * K, K)],
                             edg_v.at[b], esem[b])

        def wait_edge(ci, b):
            pltpu.make_async_copy(edge_hbm.at[pl.ds(wid * EPW + ci * K, K)],
                                  edg_v.at[b], esem[b]).wait()

        def issue_gather(b, bi):
            pltpu.async_copy(node_hbm.at[src_v.at[bi]], gat_v.at[b],
                             gsem[b])

        def wait_gather(b, bi):
            pltpu.make_async_copy(node_hbm.at[src_v.at[bi]], gat_v.at[b],
                                  gsem[b]).wait()

        def issue_scatter(b, bi):
            pltpu.async_copy(gat_v.at[b], acc_sh.at[dst_v.at[bi]], ssem[b],
                             add=True)

        def wait_scatter(b, bi):
            pltpu.make_async_copy(gat_v.at[b], acc_sh.at[dst_v.at[bi]],
                                  ssem[b]).wait()

        for pch in range(NB):
            if pch < CH:
                issue_idx(pch, pch)
                issue_edge(pch, pch)
        wait_idx(0, 0)
        issue_gather(0, 0)

        def ring(g, carry):
            for b6 in range(2 * NB):
                ci = g * 2 * NB + b6
                b = b6 % NB
                bp1 = (b + 1) % NB
                bi1 = (b6 + 1) % (2 * NB)
                bi3 = (b6 + NB) % (2 * NB)

                @pl.when(ci < CH)
                def _section():
                    wait_gather(b, b6)
                    wait_edge(ci, b)

                    @pl.when(ci + 1 < CH)
                    def _():
                        wait_idx(ci + 1, bi1)

                    @pl.when(ci >= 2)
                    def _():
                        wait_scatter(bp1, (b6 + 2 * NB - 2) % (2 * NB))

                    @pl.when(ci + 1 < CH)
                    def _():
                        issue_gather(bp1, bi1)

                    def row(r, rc):
                        for j in range(H // 16):
                            sl = pl.ds(j * 16, 16)
                            gat_v[b, r, sl] = jnp.maximum(
                                gat_v[b, r, sl] + edg_v[b, r, sl], 0.0)
                        return rc
                    lax.fori_loop(0, K, row, 0)

                    issue_scatter(b, b6)

                    @pl.when(ci + NB < CH)
                    def _():
                        issue_idx(ci + NB, bi3)
                        issue_edge(ci + NB, b)
            return carry
        lax.fori_loop(0, (CH + 2 * NB - 1) // (2 * NB), ring, 0)

        wait_scatter((CH - 2) % NB, (CH - 2) % (2 * NB))
        wait_scatter((CH - 1) % NB, (CH - 1) % (2 * NB))
        plsc.subcore_barrier()
        pltpu.sync_copy(acc_sh.at[pl.ds(base0, RPS)],
                        out_hbm.at[c].at[pl.ds(base0, RPS)])

    return msgpass


def _dot(a, b):
    return jax.lax.dot_general(a, b, (((1,), (0,)), ((), ())),
                               preferred_element_type=jnp.float32)


def _node_encode_body(x_ref, w_ref, b_ref, o_ref):
    o_ref[...] = _dot(x_ref[...], w_ref[...]) + b_ref[...]


def _edge_encode_body(ea_ref, w0_ref, b0_ref, e0_ref):
    e0_ref[...] = _dot(ea_ref[...], w0_ref[...]) + b0_ref[...]


def _edge_mlp_body(e0_ref, w1_ref, b1_ref, w2_ref, b2_ref, e1_ref):
    t = jnp.maximum(_dot(e0_ref[...], w1_ref[...]) + b1_ref[...], 0.0)
    e1_ref[...] = _dot(t, w2_ref[...]) + b2_ref[...]


def _update_body(h_ref, a0_ref, a1_ref, w1_ref, b1_ref, w2_ref, b2_ref,
                 wih_ref, bih_ref, whh_ref, bhh_ref, eps_ref, o_ref):
    h = h_ref[...]
    u = (1.0 + eps_ref[0, 0]) * h + a0_ref[0] + a1_ref[0]
    t = jnp.maximum(_dot(u, w1_ref[...]) + b1_ref[...], 0.0)
    m = _dot(t, w2_ref[...]) + b2_ref[...]
    gi = _dot(m, wih_ref[...]) + bih_ref[...]
    gh = _dot(h, whh_ref[...]) + bhh_ref[...]
    r = jax.nn.sigmoid(gi[:, 0:H] + gh[:, 0:H])
    z = jax.nn.sigmoid(gi[:, H:2 * H] + gh[:, H:2 * H])
    n = jnp.tanh(gi[:, 2 * H:3 * H] + r * gh[:, 2 * H:3 * H])
    o_ref[...] = (1.0 - z) * n + z * h


def _set2set_body(h_ref, bat_ref, wih_ref, bih_ref, whh_ref, bhh_ref,
                  f1w_ref, f1b_ref, f2w_ref, f2b_ref, o_ref):
    h = h_ref[...]
    seg = jax.lax.broadcasted_iota(jnp.int32, (B, NP), 0)
    mask = bat_ref[...] == seg
    hs = jnp.zeros((B, H), jnp.float32)
    cs = jnp.zeros((B, H), jnp.float32)
    qs = jnp.zeros((B, 2 * H), jnp.float32)
    for _ in range(3):
        gates = (_dot(qs, wih_ref[...]) + bih_ref[...]
                 + _dot(hs, whh_ref[...]) + bhh_ref[...])
        gi = gates[:, 0:H]
        gf = gates[:, H:2 * H]
        gg = gates[:, 2 * H:3 * H]
        go = gates[:, 3 * H:4 * H]
        cs = jax.nn.sigmoid(gf) * cs + jax.nn.sigmoid(gi) * jnp.tanh(gg)
        hs = jax.nn.sigmoid(go) * jnp.tanh(cs)
        st = jax.lax.dot_general(hs, h, (((1,), (1,)), ((), ())),
                                 preferred_element_type=jnp.float32)
        sm = jnp.where(mask, st, -jnp.inf)
        e_max = jnp.maximum(jnp.max(sm, axis=1, keepdims=True), -1e30)
        a = jnp.where(mask, jnp.exp(st - e_max), 0.0)
        denom = jnp.sum(a, axis=1, keepdims=True)
        attn = a / (denom + 1e-16)
        r = _dot(attn, h)
        qs = jnp.concatenate([hs, r], axis=1)
    z1 = jnp.maximum(_dot(qs, f1w_ref[...]) + f1b_ref[...], 0.0)
    o_ref[...] = _dot(z1, f2w_ref[...]) + f2b_ref[...]


def _full(shape):
    return pl.BlockSpec(shape, lambda i: (0, 0))


def _rows(blk, d):
    return pl.BlockSpec((blk, d), lambda i: (i, 0))


def kernel(x, edge_index, edge_attr, batch, params):
    f32 = jnp.float32
    ei = edge_index.astype(jnp.int32)
    src2 = ei[0].reshape(NW, CH, K)
    dst2 = ei[1].reshape(NW, CH, K)
    x_p = jnp.pad(x, ((0, NP - N), (0, 0)))
    bat_p = jnp.pad(batch.astype(jnp.int32), (0, NP - N),
                    constant_values=B).reshape(1, NP)

    p = params
    row1 = lambda a: a.reshape(1, -1)

    node0 = pl.pallas_call(
        _node_encode_body,
        grid=(NP // 2048,),
        in_specs=[_rows(2048, DN), _full((DN, H)), _full((1, H))],
        out_specs=_rows(2048, H),
        out_shape=jax.ShapeDtypeStruct((NP, H), f32),
    )(x_p, p["node_lin0"]["W"], row1(p["node_lin0"]["b"]))

    eblk = 2000
    e0 = pl.pallas_call(
        _edge_encode_body,
        grid=(E // eblk,),
        in_specs=[_rows(eblk, DE), _full((DE, H)), _full((1, H))],
        out_specs=_rows(eblk, H),
        out_shape=jax.ShapeDtypeStruct((E, H), f32),
    )(edge_attr, p["edge_lin0"]["W"], row1(p["edge_lin0"]["b"]))

    gru = p["gru"]
    wih_t = gru["W_ih"].T
    whh_t = gru["W_hh"].T
    bih = row1(gru["b_ih"])
    bhh = row1(gru["b_hh"])

    def update(h, aggs, gp):
        ublk = 2048
        return pl.pallas_call(
            _update_body,
            grid=(NP // ublk,),
            in_specs=[_rows(ublk, H),
                      pl.BlockSpec((1, ublk, H), lambda i: (0, i, 0)),
                      pl.BlockSpec((1, ublk, H), lambda i: (1, i, 0)),
                      _full((H, H)), _full((1, H)), _full((H, H)),
                      _full((1, H)), _full((H, 3 * H)), _full((1, 3 * H)),
                      _full((H, 3 * H)), _full((1, 3 * H)),
                      pl.BlockSpec(memory_space=pltpu.SMEM)],
            out_specs=_rows(ublk, H),
            out_shape=jax.ShapeDtypeStruct((NP, H), f32),
        )(h, aggs, aggs,
          gp["mlp1"]["W"], row1(gp["mlp1"]["b"]),
          gp["mlp2"]["W"], row1(gp["mlp2"]["b"]),
          wih_t, bih, whh_t, bhh, gp["eps"].reshape(1, 1))

    aggs0 = _make_msgpass()(node0, e0, src2, dst2)
    e1 = pl.pallas_call(
        _edge_mlp_body,
        grid=(E // eblk,),
        in_specs=[_rows(eblk, H), _full((H, H)), _full((1, H)),
                  _full((H, H)), _full((1, H))],
        out_specs=_rows(eblk, H),
        out_shape=jax.ShapeDtypeStruct((E, H), f32),
    )(e0, p["edge_mlp"][0]["mlp1"]["W"], row1(p["edge_mlp"][0]["mlp1"]["b"]),
      p["edge_mlp"][0]["mlp2"]["W"], row1(p["edge_mlp"][0]["mlp2"]["b"]))
    h = update(node0, aggs0, p["gin"][0])

    aggs1 = _make_msgpass()(h, e1, src2, dst2)
    h = update(h, aggs1, p["gin"][1])

    lstm = p["lstm"]
    f2w = jnp.zeros((H, H), f32).at[:, 0].set(p["fc2"]["W"][:, 0])
    f2b = jnp.zeros((1, H), f32).at[0, 0].set(p["fc2"]["b"][0])
    out = pl.pallas_call(
        _set2set_body,
        grid=(1,),
        in_specs=[_rows(NP, H), _full((1, NP)),
                  _full((2 * H, 4 * H)), _full((1, 4 * H)),
                  _full((H, 4 * H)), _full((1, 4 * H)),
                  _full((2 * H, H)), _full((1, H)),
                  _full((H, H)), _full((1, H))],
        out_specs=_rows(B, H),
        out_shape=jax.ShapeDtypeStruct((B, H), f32),
    )(h, bat_p, lstm["W_ih"].T, row1(lstm["b_ih"]),
      lstm["W_hh"].T, row1(lstm["b_hh"]),
      p["fc1"]["W"], row1(p["fc1"]["b"]), f2w, f2b)
    return out[:, 0]


# speedup vs baseline: 4.8210x; 1.0480x over previous
import functools

import jax
import jax.numpy as jnp
from jax import lax
from jax.experimental import pallas as pl
from jax.experimental.pallas import tpu as pltpu
from jax.experimental.pallas import tpu_sc as plsc

N = 10000
E = 320000
DN = 128
DE = 16
H = 128
B = 16

NP = 10240
NW = 32
K = 40
CH = E // (NW * K)
EPW = E // NW
RPS = NP // 16
NB = 3


@functools.lru_cache(maxsize=None)
def _make_msgpass():
    mesh = plsc.VectorSubcoreMesh(core_axis_name="c", subcore_axis_name="s")

    @functools.partial(
        pl.kernel,
        mesh=mesh,
        out_type=jax.ShapeDtypeStruct((2, NP, H), jnp.float32),
        scratch_types=[
            pltpu.VMEM((2 * NB, K), jnp.int32),
            pltpu.VMEM((2 * NB, K), jnp.int32),
            pltpu.VMEM((NB, K, H), jnp.float32),
            pltpu.VMEM((NB, K, H), jnp.float32),
            pltpu.VMEM_SHARED((NP, H), jnp.float32),
        ] + [pltpu.SemaphoreType.DMA] * (4 * NB),
    )
    def msgpass(node_hbm, edge_hbm, src_hbm, dst_hbm, out_hbm,
                src_v, dst_v, gat_v, edg_v, acc_sh, *sems):
        c = lax.axis_index("c")
        s = lax.axis_index("s")
        wid = s * 2 + c
        gsem = sems[0:NB]
        esem = sems[NB:2 * NB]
        isem = sems[2 * NB:3 * NB]
        ssem = sems[3 * NB:4 * NB]

        def zrow(r, carry):
            for j in range(H // 16):
                gat_v[0, r, pl.ds(j * 16, 16)] = jnp.zeros((16,),
                                                           jnp.float32)
            return carry
        lax.fori_loop(0, K, zrow, 0)
        base0 = s * RPS
        for i in range(RPS // K):
            pltpu.sync_copy(gat_v.at[0], acc_sh.at[pl.ds(base0 + i * K, K)])
        rem = RPS - (RPS // K) * K
        if rem:
            pltpu.sync_copy(gat_v.at[0].at[pl.ds(0, rem)],
                            acc_sh.at[pl.ds(base0 + (RPS // K) * K, rem)])
        plsc.subcore_barrier()

        def issue_idx(ci, bi):
            pltpu.async_copy(src_hbm.at[wid, ci], src_v.at[bi],
                             isem[bi % NB])
            pltpu.async_copy(dst_hbm.at[wid, ci], dst_v.at[bi],
                             isem[bi % NB])

        def wait_idx(ci, bi):
            pltpu.make_async_copy(src_hbm.at[wid, ci], src_v.at[bi],
                                  isem[bi % NB]).wait()
            pltpu.make_async_copy(dst_hbm.at[wid, ci], dst_v.at[bi],
                                  isem[bi % NB]).wait()

        def issue_edge(ci, b):
            pltpu.async_copy(edge_hbm.at[pl.ds(wid * EPW + ci * K, K)],
                             edg_v.at[b], esem[b])

        def wait_edge(ci, b):
            pltpu.make_async_copy(edge_hbm.at[pl.ds(wid * EPW + ci * K, K)],
                                  edg_v.at[b], esem[b]).wait()

        def issue_gather(b, bi):
            pltpu.async_copy(node_hbm.at[src_v.at[bi]], gat_v.at[b],
                             gsem[b])

        def wait_gather(b, bi):
            pltpu.make_async_copy(node_hbm.at[src_v.at[bi]], gat_v.at[b],
                                  gsem[b]).wait()

        def issue_scatter(b, bi):
            pltpu.async_copy(gat_v.at[b], acc_sh.at[dst_v.at[bi]], ssem[b],
                             add=True)

        def wait_scatter(b, bi):
            pltpu.make_async_copy(gat_v.at[b], acc_sh.at[dst_v.at[bi]],
                                  ssem[b]).wait()

        for pch in range(NB):
            if pch < CH:
                issue_idx(pch, pch)
                issue_edge(pch, pch)
        wait_idx(0, 0)
        issue_gather(0, 0)

        def ring(g, carry):
            for b6 in range(2 * NB):
                ci = g * 2 * NB + b6
                b = b6 % NB
                bp1 = (b + 1) % NB
                bi1 = (b6 + 1) % (2 * NB)
                bi3 = (b6 + NB) % (2 * NB)

                @pl.when(ci < CH)
                def _section():
                    wait_gather(b, b6)
                    wait_edge(ci, b)

                    @pl.when(ci + 1 < CH)
                    def _():
                        wait_idx(ci + 1, bi1)

                    @pl.when(ci >= 2)
                    def _():
                        wait_scatter(bp1, (b6 + 2 * NB - 2) % (2 * NB))

                    @pl.when(ci + 1 < CH)
                    def _():
                        issue_gather(bp1, bi1)

                    def row(r, rc):
                        for j in range(H // 16):
                            sl = pl.ds(j * 16, 16)
                            gat_v[b, r, sl] = jnp.maximum(
                                gat_v[b, r, sl] + edg_v[b, r, sl], 0.0)
                        return rc
                    lax.fori_loop(0, K, row, 0)

                    issue_scatter(b, b6)

                    @pl.when(ci + NB < CH)
                    def _():
                        issue_idx(ci + NB, bi3)
                        issue_edge(ci + NB, b)
            return carry
        lax.fori_loop(0, (CH + 2 * NB - 1) // (2 * NB), ring, 0)

        wait_scatter((CH - 2) % NB, (CH - 2) % (2 * NB))
        wait_scatter((CH - 1) % NB, (CH - 1) % (2 * NB))
        plsc.subcore_barrier()
        pltpu.sync_copy(acc_sh.at[pl.ds(base0, RPS)],
                        out_hbm.at[c].at[pl.ds(base0, RPS)])

    return msgpass


def _dot(a, b):
    return jax.lax.dot_general(a, b, (((1,), (0,)), ((), ())),
                               preferred_element_type=jnp.float32)


def _node_encode_body(x_ref, w_ref, b_ref, o_ref):
    o_ref[...] = _dot(x_ref[...], w_ref[...]) + b_ref[...]


def _edge_encode_body(ea_ref, w0_ref, b0_ref, e0_ref):
    e0_ref[...] = _dot(ea_ref[...], w0_ref[...]) + b0_ref[...]


def _edge_enc_mlp_body(ea_ref, w0_ref, b0_ref, w1_ref, b1_ref, w2_ref,
                       b2_ref, e1_ref):
    e0 = _dot(ea_ref[...], w0_ref[...]) + b0_ref[...]
    t = jnp.maximum(_dot(e0, w1_ref[...]) + b1_ref[...], 0.0)
    e1_ref[...] = _dot(t, w2_ref[...]) + b2_ref[...]


def _update_body(h_ref, a0_ref, a1_ref, w1_ref, b1_ref, w2_ref, b2_ref,
                 wih_ref, bih_ref, whh_ref, bhh_ref, eps_ref, o_ref):
    h = h_ref[...]
    u = (1.0 + eps_ref[0, 0]) * h + a0_ref[0] + a1_ref[0]
    t = jnp.maximum(_dot(u, w1_ref[...]) + b1_ref[...], 0.0)
    m = _dot(t, w2_ref[...]) + b2_ref[...]
    gi = _dot(m, wih_ref[...]) + bih_ref[...]
    gh = _dot(h, whh_ref[...]) + bhh_ref[...]
    r = jax.nn.sigmoid(gi[:, 0:H] + gh[:, 0:H])
    z = jax.nn.sigmoid(gi[:, H:2 * H] + gh[:, H:2 * H])
    n = jnp.tanh(gi[:, 2 * H:3 * H] + r * gh[:, 2 * H:3 * H])
    o_ref[...] = (1.0 - z) * n + z * h


def _set2set_body(h_ref, bat_ref, wih_ref, bih_ref, whh_ref, bhh_ref,
                  f1w_ref, f1b_ref, f2w_ref, f2b_ref, o_ref):
    h = h_ref[...]
    seg = jax.lax.broadcasted_iota(jnp.int32, (B, NP), 0)
    mask = bat_ref[...] == seg
    hs = jnp.zeros((B, H), jnp.float32)
    cs = jnp.zeros((B, H), jnp.float32)
    qs = jnp.zeros((B, 2 * H), jnp.float32)
    for _ in range(3):
        gates = (_dot(qs, wih_ref[...]) + bih_ref[...]
                 + _dot(hs, whh_ref[...]) + bhh_ref[...])
        gi = gates[:, 0:H]
        gf = gates[:, H:2 * H]
        gg = gates[:, 2 * H:3 * H]
        go = gates[:, 3 * H:4 * H]
        cs = jax.nn.sigmoid(gf) * cs + jax.nn.sigmoid(gi) * jnp.tanh(gg)
        hs = jax.nn.sigmoid(go) * jnp.tanh(cs)
        st = jax.lax.dot_general(hs, h, (((1,), (1,)), ((), ())),
                                 preferred_element_type=jnp.float32)
        sm = jnp.where(mask, st, -jnp.inf)
        e_max = jnp.maximum(jnp.max(sm, axis=1, keepdims=True), -1e30)
        a = jnp.where(mask, jnp.exp(st - e_max), 0.0)
        denom = jnp.sum(a, axis=1, keepdims=True)
        attn = a / (denom + 1e-16)
        r = _dot(attn, h)
        qs = jnp.concatenate([hs, r], axis=1)
    z1 = jnp.maximum(_dot(qs, f1w_ref[...]) + f1b_ref[...], 0.0)
    o_ref[...] = _dot(z1, f2w_ref[...]) + f2b_ref[...]


def _full(shape):
    return pl.BlockSpec(shape, lambda i: (0, 0))


def _rows(blk, d):
    return pl.BlockSpec((blk, d), lambda i: (i, 0))


def kernel(x, edge_index, edge_attr, batch, params):
    f32 = jnp.float32
    ei = edge_index.astype(jnp.int32)
    src2 = ei[0].reshape(NW, CH, K)
    dst2 = ei[1].reshape(NW, CH, K)
    x_p = jnp.pad(x, ((0, NP - N), (0, 0)))
    bat_p = jnp.pad(batch.astype(jnp.int32), (0, NP - N),
                    constant_values=B).reshape(1, NP)

    p = params
    row1 = lambda a: a.reshape(1, -1)

    node0 = pl.pallas_call(
        _node_encode_body,
        grid=(NP // 2048,),
        in_specs=[_rows(2048, DN), _full((DN, H)), _full((1, H))],
        out_specs=_rows(2048, H),
        out_shape=jax.ShapeDtypeStruct((NP, H), f32),
    )(x_p, p["node_lin0"]["W"], row1(p["node_lin0"]["b"]))

    eblk = 4000
    e0 = pl.pallas_call(
        _edge_encode_body,
        grid=(E // eblk,),
        in_specs=[_rows(eblk, DE), _full((DE, H)), _full((1, H))],
        out_specs=_rows(eblk, H),
        out_shape=jax.ShapeDtypeStruct((E, H), f32),
    )(edge_attr, p["edge_lin0"]["W"], row1(p["edge_lin0"]["b"]))

    gru = p["gru"]
    wih_t = gru["W_ih"].T
    whh_t = gru["W_hh"].T
    bih = row1(gru["b_ih"])
    bhh = row1(gru["b_hh"])

    def update(h, aggs, gp):
        ublk = 2048
        return pl.pallas_call(
            _update_body,
            grid=(NP // ublk,),
            in_specs=[_rows(ublk, H),
                      pl.BlockSpec((1, ublk, H), lambda i: (0, i, 0)),
                      pl.BlockSpec((1, ublk, H), lambda i: (1, i, 0)),
                      _full((H, H)), _full((1, H)), _full((H, H)),
                      _full((1, H)), _full((H, 3 * H)), _full((1, 3 * H)),
                      _full((H, 3 * H)), _full((1, 3 * H)),
                      pl.BlockSpec(memory_space=pltpu.SMEM)],
            out_specs=_rows(ublk, H),
            out_shape=jax.ShapeDtypeStruct((NP, H), f32),
        )(h, aggs, aggs,
          gp["mlp1"]["W"], row1(gp["mlp1"]["b"]),
          gp["mlp2"]["W"], row1(gp["mlp2"]["b"]),
          wih_t, bih, whh_t, bhh, gp["eps"].reshape(1, 1))

    aggs0 = _make_msgpass()(node0, e0, src2, dst2)
    e1 = pl.pallas_call(
        _edge_enc_mlp_body,
        grid=(E // eblk,),
        in_specs=[_rows(eblk, DE), _full((DE, H)), _full((1, H)),
                  _full((H, H)), _full((1, H)),
                  _full((H, H)), _full((1, H))],
        out_specs=_rows(eblk, H),
        out_shape=jax.ShapeDtypeStruct((E, H), f32),
    )(edge_attr, p["edge_lin0"]["W"], row1(p["edge_lin0"]["b"]),
      p["edge_mlp"][0]["mlp1"]["W"], row1(p["edge_mlp"][0]["mlp1"]["b"]),
      p["edge_mlp"][0]["mlp2"]["W"], row1(p["edge_mlp"][0]["mlp2"]["b"]))
    h = update(node0, aggs0, p["gin"][0])

    aggs1 = _make_msgpass()(h, e1, src2, dst2)
    h = update(h, aggs1, p["gin"][1])

    lstm = p["lstm"]
    f2w = jnp.zeros((H, H), f32).at[:, 0].set(p["fc2"]["W"][:, 0])
    f2b = jnp.zeros((1, H), f32).at[0, 0].set(p["fc2"]["b"][0])
    out = pl.pallas_call(
        _set2set_body,
        grid=(1,),
        in_specs=[_rows(NP, H), _full((1, NP)),
                  _full((2 * H, 4 * H)), _full((1, 4 * H)),
                  _full((H, 4 * H)), _full((1, 4 * H)),
                  _full((2 * H, H)), _full((1, H)),
                  _full((H, H)), _full((1, H))],
        out_specs=_rows(B, H),
        out_shape=jax.ShapeDtypeStruct((B, H), f32),
    )(h, bat_p, lstm["W_ih"].T, row1(lstm["b_ih"]),
      lstm["W_hh"].T, row1(lstm["b_hh"]),
      p["fc1"]["W"], row1(p["fc1"]["b"]), f2w, f2b)
    return out[:, 0]


# speedup vs baseline: 5.9033x; 1.2245x over previous
import functools

import jax
import jax.numpy as jnp
from jax import lax
from jax.experimental import pallas as pl
from jax.experimental.pallas import tpu as pltpu
from jax.experimental.pallas import tpu_sc as plsc

N = 10000
E = 320000
DN = 128
DE = 16
H = 128
B = 16

NP = 10240
NW = 32
K = 40
CH = E // (NW * K)
EPW = E // NW
RPS = NP // 16
NB = 3


@functools.lru_cache(maxsize=None)
def _make_msgpass():
    mesh = plsc.VectorSubcoreMesh(core_axis_name="c", subcore_axis_name="s")

    @functools.partial(
        pl.kernel,
        mesh=mesh,
        out_type=jax.ShapeDtypeStruct((2, NP, H), jnp.float32),
        scratch_types=[
            pltpu.VMEM((2 * NB, K), jnp.int32),
            pltpu.VMEM((2 * NB, K), jnp.int32),
            pltpu.VMEM((NB, K, H), jnp.float32),
            pltpu.VMEM((NB, K, H), jnp.float32),
            pltpu.VMEM_SHARED((NP, H), jnp.float32),
        ] + [pltpu.SemaphoreType.DMA] * (4 * NB),
    )
    def msgpass(node_hbm, edge_hbm, src_hbm, dst_hbm, out_hbm,
                src_v, dst_v, gat_v, edg_v, acc_sh, *sems):
        c = lax.axis_index("c")
        s = lax.axis_index("s")
        wid = s * 2 + c
        gsem = sems[0:NB]
        esem = sems[NB:2 * NB]
        isem = sems[2 * NB:3 * NB]
        ssem = sems[3 * NB:4 * NB]

        def zrow(r, carry):
            for j in range(H // 16):
                gat_v[0, r, pl.ds(j * 16, 16)] = jnp.zeros((16,),
                                                           jnp.float32)
            return carry
        lax.fori_loop(0, K, zrow, 0)
        base0 = s * RPS
        for i in range(RPS // K):
            pltpu.sync_copy(gat_v.at[0], acc_sh.at[pl.ds(base0 + i * K, K)])
        rem = RPS - (RPS // K) * K
        if rem:
            pltpu.sync_copy(gat_v.at[0].at[pl.ds(0, rem)],
                            acc_sh.at[pl.ds(base0 + (RPS // K) * K, rem)])
        plsc.subcore_barrier()

        def issue_idx(ci, bi):
            pltpu.async_copy(src_hbm.at[wid, ci], src_v.at[bi],
                             isem[bi % NB])
            pltpu.async_copy(dst_hbm.at[wid, ci], dst_v.at[bi],
                             isem[bi % NB])

        def wait_idx(ci, bi):
            pltpu.make_async_copy(src_hbm.at[wid, ci], src_v.at[bi],
                                  isem[bi % NB]).wait()
            pltpu.make_async_copy(dst_hbm.at[wid, ci], dst_v.at[bi],
                                  isem[bi % NB]).wait()

        def issue_edge(ci, b):
            pltpu.async_copy(edge_hbm.at[pl.ds(wid * EPW + ci * K, K)],
                             edg_v.at[b], esem[b])

        def wait_edge(ci, b):
            pltpu.make_async_copy(edge_hbm.at[pl.ds(wid * EPW + ci * K, K)],
                                  edg_v.at[b], esem[b]).wait()

        def issue_gather(b, bi):
            pltpu.async_copy(node_hbm.at[src_v.at[bi]], gat_v.at[b],
                             gsem[b])

        def wait_gather(b, bi):
            pltpu.make_async_copy(node_hbm.at[src_v.at[bi]], gat_v.at[b],
                                  gsem[b]).wait()

        def issue_scatter(b, bi):
            pltpu.async_copy(gat_v.at[b], acc_sh.at[dst_v.at[bi]], ssem[b],
                             add=True)

        def wait_scatter(b, bi):
            pltpu.make_async_copy(gat_v.at[b], acc_sh.at[dst_v.at[bi]],
                                  ssem[b]).wait()

        for pch in range(NB):
            if pch < CH:
                issue_idx(pch, pch)
                issue_edge(pch, pch)
        wait_idx(0, 0)
        issue_gather(0, 0)

        def ring(g, carry):
            for b6 in range(2 * NB):
                ci = g * 2 * NB + b6
                b = b6 % NB
                bp1 = (b + 1) % NB
                bi1 = (b6 + 1) % (2 * NB)
                bi3 = (b6 + NB) % (2 * NB)

                @pl.when(ci < CH)
                def _section():
                    wait_gather(b, b6)
                    wait_edge(ci, b)

                    @pl.when(ci + 1 < CH)
                    def _():
                        wait_idx(ci + 1, bi1)

                    @pl.when(ci >= 2)
                    def _():
                        wait_scatter(bp1, (b6 + 2 * NB - 2) % (2 * NB))

                    @pl.when(ci + 1 < CH)
                    def _():
                        issue_gather(bp1, bi1)

                    def row(r, rc):
                        for j in range(H // 16):
                            sl = pl.ds(j * 16, 16)
                            gat_v[b, r, sl] = jnp.maximum(
                                gat_v[b, r, sl] + edg_v[b, r, sl], 0.0)
                        return rc
                    lax.fori_loop(0, K, row, 0)

                    issue_scatter(b, b6)

                    @pl.when(ci + NB < CH)
                    def _():
                        issue_idx(ci + NB, bi3)
                        issue_edge(ci + NB, b)
            return carry
        lax.fori_loop(0, (CH + 2 * NB - 1) // (2 * NB), ring, 0)

        wait_scatter((CH - 2) % NB, (CH - 2) % (2 * NB))
        wait_scatter((CH - 1) % NB, (CH - 1) % (2 * NB))
        plsc.subcore_barrier()
        pltpu.sync_copy(acc_sh.at[pl.ds(base0, RPS)],
                        out_hbm.at[c].at[pl.ds(base0, RPS)])

    return msgpass


def _dot(a, b):
    return jax.lax.dot_general(a, b, (((1,), (0,)), ((), ())),
                               preferred_element_type=jnp.float32)


def _node_encode_body(x_ref, w_ref, b_ref, o_ref):
    o_ref[...] = _dot(x_ref[...], w_ref[...]) + b_ref[...]


def _dot_t(a, b):
    return jax.lax.dot_general(a, b, (((0,), (0,)), ((), ())),
                               preferred_element_type=jnp.float32)


def _edge_encode_body(ea_ref, w0_ref, b0_ref, e0_ref):
    e0_ref[...] = _dot_t(ea_ref[...], w0_ref[...]) + b0_ref[...]


def _edge_enc_mlp_body(ea_ref, w0_ref, b0_ref, w1_ref, b1_ref, w2_ref,
                       b2_ref, e1_ref):
    e0 = _dot_t(ea_ref[...], w0_ref[...]) + b0_ref[...]
    t = jnp.maximum(_dot(e0, w1_ref[...]) + b1_ref[...], 0.0)
    e1_ref[...] = _dot(t, w2_ref[...]) + b2_ref[...]


def _update_body(h_ref, a0_ref, a1_ref, w1_ref, b1_ref, w2_ref, b2_ref,
                 wih_ref, bih_ref, whh_ref, bhh_ref, eps_ref, o_ref):
    h = h_ref[...]
    u = (1.0 + eps_ref[0, 0]) * h + a0_ref[0] + a1_ref[0]
    t = jnp.maximum(_dot(u, w1_ref[...]) + b1_ref[...], 0.0)
    m = _dot(t, w2_ref[...]) + b2_ref[...]
    gi = _dot(m, wih_ref[...]) + bih_ref[...]
    gh = _dot(h, whh_ref[...]) + bhh_ref[...]
    r = jax.nn.sigmoid(gi[:, 0:H] + gh[:, 0:H])
    z = jax.nn.sigmoid(gi[:, H:2 * H] + gh[:, H:2 * H])
    n = jnp.tanh(gi[:, 2 * H:3 * H] + r * gh[:, 2 * H:3 * H])
    o_ref[...] = (1.0 - z) * n + z * h


def _set2set_body(h_ref, bat_ref, wih_ref, bih_ref, whh_ref, bhh_ref,
                  f1w_ref, f1b_ref, f2w_ref, f2b_ref, o_ref):
    h = h_ref[...]
    seg = jax.lax.broadcasted_iota(jnp.int32, (B, NP), 0)
    mask = bat_ref[...] == seg
    hs = jnp.zeros((B, H), jnp.float32)
    cs = jnp.zeros((B, H), jnp.float32)
    qs = jnp.zeros((B, 2 * H), jnp.float32)
    for _ in range(3):
        gates = (_dot(qs, wih_ref[...]) + bih_ref[...]
                 + _dot(hs, whh_ref[...]) + bhh_ref[...])
        gi = gates[:, 0:H]
        gf = gates[:, H:2 * H]
        gg = gates[:, 2 * H:3 * H]
        go = gates[:, 3 * H:4 * H]
        cs = jax.nn.sigmoid(gf) * cs + jax.nn.sigmoid(gi) * jnp.tanh(gg)
        hs = jax.nn.sigmoid(go) * jnp.tanh(cs)
        st = jax.lax.dot_general(hs, h, (((1,), (1,)), ((), ())),
                                 preferred_element_type=jnp.float32)
        sm = jnp.where(mask, st, -jnp.inf)
        e_max = jnp.maximum(jnp.max(sm, axis=1, keepdims=True), -1e30)
        a = jnp.where(mask, jnp.exp(st - e_max), 0.0)
        denom = jnp.sum(a, axis=1, keepdims=True)
        attn = a / (denom + 1e-16)
        r = _dot(attn, h)
        qs = jnp.concatenate([hs, r], axis=1)
    z1 = jnp.maximum(_dot(qs, f1w_ref[...]) + f1b_ref[...], 0.0)
    o_ref[...] = _dot(z1, f2w_ref[...]) + f2b_ref[...]


def _full(shape):
    return pl.BlockSpec(shape, lambda i: (0, 0))


def _rows(blk, d):
    return pl.BlockSpec((blk, d), lambda i: (i, 0))


def kernel(x, edge_index, edge_attr, batch, params):
    f32 = jnp.float32
    ei = edge_index.astype(jnp.int32)
    src2 = ei[0].reshape(NW, CH, K)
    dst2 = ei[1].reshape(NW, CH, K)
    x_p = jnp.pad(x, ((0, NP - N), (0, 0)))
    bat_p = jnp.pad(batch.astype(jnp.int32), (0, NP - N),
                    constant_values=B).reshape(1, NP)

    p = params
    row1 = lambda a: a.reshape(1, -1)

    node0 = pl.pallas_call(
        _node_encode_body,
        grid=(NP // 2048,),
        in_specs=[_rows(2048, DN), _full((DN, H)), _full((1, H))],
        out_specs=_rows(2048, H),
        out_shape=jax.ShapeDtypeStruct((NP, H), f32),
    )(x_p, p["node_lin0"]["W"], row1(p["node_lin0"]["b"]))

    eblk = 6400
    ea_t = edge_attr.T
    ecol = pl.BlockSpec((DE, eblk), lambda i: (0, i))
    e0 = pl.pallas_call(
        _edge_encode_body,
        grid=(E // eblk,),
        in_specs=[ecol, _full((DE, H)), _full((1, H))],
        out_specs=_rows(eblk, H),
        out_shape=jax.ShapeDtypeStruct((E, H), f32),
    )(ea_t, p["edge_lin0"]["W"], row1(p["edge_lin0"]["b"]))

    gru = p["gru"]
    wih_t = gru["W_ih"].T
    whh_t = gru["W_hh"].T
    bih = row1(gru["b_ih"])
    bhh = row1(gru["b_hh"])

    def update(h, aggs, gp):
        ublk = 2048
        return pl.pallas_call(
            _update_body,
            grid=(NP // ublk,),
            in_specs=[_rows(ublk, H),
                      pl.BlockSpec((1, ublk, H), lambda i: (0, i, 0)),
                      pl.BlockSpec((1, ublk, H), lambda i: (1, i, 0)),
                      _full((H, H)), _full((1, H)), _full((H, H)),
                      _full((1, H)), _full((H, 3 * H)), _full((1, 3 * H)),
                      _full((H, 3 * H)), _full((1, 3 * H)),
                      pl.BlockSpec(memory_space=pltpu.SMEM)],
            out_specs=_rows(ublk, H),
            out_shape=jax.ShapeDtypeStruct((NP, H), f32),
        )(h, aggs, aggs,
          gp["mlp1"]["W"], row1(gp["mlp1"]["b"]),
          gp["mlp2"]["W"], row1(gp["mlp2"]["b"]),
          wih_t, bih, whh_t, bhh, gp["eps"].reshape(1, 1))

    aggs0 = _make_msgpass()(node0, e0, src2, dst2)
    e1 = pl.pallas_call(
        _edge_enc_mlp_body,
        grid=(E // eblk,),
        in_specs=[ecol, _full((DE, H)), _full((1, H)),
                  _full((H, H)), _full((1, H)),
                  _full((H, H)), _full((1, H))],
        out_specs=_rows(eblk, H),
        out_shape=jax.ShapeDtypeStruct((E, H), f32),
    )(ea_t, p["edge_lin0"]["W"], row1(p["edge_lin0"]["b"]),
      p["edge_mlp"][0]["mlp1"]["W"], row1(p["edge_mlp"][0]["mlp1"]["b"]),
      p["edge_mlp"][0]["mlp2"]["W"], row1(p["edge_mlp"][0]["mlp2"]["b"]))
    h = update(node0, aggs0, p["gin"][0])

    aggs1 = _make_msgpass()(h, e1, src2, dst2)
    h = update(h, aggs1, p["gin"][1])

    lstm = p["lstm"]
    f2w = jnp.zeros((H, H), f32).at[:, 0].set(p["fc2"]["W"][:, 0])
    f2b = jnp.zeros((1, H), f32).at[0, 0].set(p["fc2"]["b"][0])
    out = pl.pallas_call(
        _set2set_body,
        grid=(1,),
        in_specs=[_rows(NP, H), _full((1, NP)),
                  _full((2 * H, 4 * H)), _full((1, 4 * H)),
                  _full((H, 4 * H)), _full((1, 4 * H)),
                  _full((2 * H, H)), _full((1, H)),
                  _full((H, H)), _full((1, H))],
        out_specs=_rows(B, H),
        out_shape=jax.ShapeDtypeStruct((B, H), f32),
    )(h, bat_p, lstm["W_ih"].T, row1(lstm["b_ih"]),
      lstm["W_hh"].T, row1(lstm["b_hh"]),
      p["fc1"]["W"], row1(p["fc1"]["b"]), f2w, f2b)
    return out[:, 0]
